# baseline (speedup 1.0000x reference)
import jax
import jax.numpy as jnp
from jax import lax
from jax.experimental import pallas as pl
from jax.experimental.pallas import tpu as pltpu
from jax.experimental.pallas import tpu_sc as plsc

N = 100000
E = 1600000
D = 128
ROW = 144
K = 7168
NBUCKET = 14
NPAD = NBUCKET * K
NTILES = 16
CHUNK = 1024
NCHUNK = 98
EPT = NCHUNK * CHUNK
EPAD = NTILES * EPT
ACC_ROWS = K + 16
PEND = 2176
FIRE_T = 1024
NEG = -1e30
NBLK = 2000
EBLK = 8000


def _enc_body(x_ref, encW_ref, encb_ref, bg_ref, bb_ref, bm_ref, bv_ref,
              W_ref, as_ref, ad_ref, xinit_ref, T_ref, adcol_ref):
    h = jnp.dot(x_ref[...], encW_ref[...], preferred_element_type=jnp.float32)
    h = h + encb_ref[...]
    h = (h - bm_ref[...]) / jnp.sqrt(bv_ref[...] + 1e-5) * bg_ref[...] + bb_ref[...]
    h = jnp.maximum(h, 0.0)
    xinit_ref[...] = h
    xt = jnp.dot(h, W_ref[...], preferred_element_type=jnp.float32)
    T_ref[:, :D] = xt
    a_s = jnp.dot(xt, as_ref[...], preferred_element_type=jnp.float32)
    T_ref[:, D:ROW] = jnp.broadcast_to(a_s[:, None], (NBLK, ROW - D))
    adcol_ref[...] = jnp.dot(xt, ad_ref[...], preferred_element_type=jnp.float32)[:, None]


def _fin_body(acc_ref, bias_ref, W_ref, as_ref, ad_ref, T_ref, adcol_ref):
    den = acc_ref[:, D:D + 1] + 1e-16
    h = acc_ref[:, :D] / den + bias_ref[...]
    h = jnp.maximum(h, 0.0)
    xt = jnp.dot(h, W_ref[...], preferred_element_type=jnp.float32)
    T_ref[:, :D] = xt
    a_s = jnp.dot(xt, as_ref[...], preferred_element_type=jnp.float32)
    T_ref[:, D:ROW] = jnp.broadcast_to(a_s[:, None], (NBLK, ROW - D))
    adcol_ref[...] = jnp.dot(xt, ad_ref[...], preferred_element_type=jnp.float32)[:, None]


def _last_body(acc_ref, bias_ref, xinit_ref, out_ref):
    den = acc_ref[:, D:D + 1] + 1e-16
    out_ref[...] = acc_ref[:, :D] / den + bias_ref[...] + xinit_ref[...]


def _el_body(ea_ref, We_ref, ae_ref, el_ref):
    m = jnp.dot(We_ref[...], ae_ref[...], preferred_element_type=jnp.float32)
    el_ref[...] = jnp.dot(ea_ref[...], m, preferred_element_type=jnp.float32)[:, None]


def _full2d(shape):
    return pl.BlockSpec(shape, lambda i: (0,) * len(shape))


def _enc_call(x, enc_W, enc_b, bn_g, bn_b, bn_m, bn_v, W, a_s, a_d):
    grid = (N // NBLK,)
    return pl.pallas_call(
        _enc_body,
        grid=grid,
        in_specs=[
            pl.BlockSpec((NBLK, 8), lambda i: (i, 0)),
            _full2d((8, D)), _full2d((D,)), _full2d((D,)), _full2d((D,)),
            _full2d((D,)), _full2d((D,)),
            _full2d((D, D)), _full2d((D,)), _full2d((D,)),
        ],
        out_specs=[
            pl.BlockSpec((NBLK, D), lambda i: (i, 0)),
            pl.BlockSpec((NBLK, ROW), lambda i: (i, 0)),
            pl.BlockSpec((NBLK, 1), lambda i: (i, 0)),
        ],
        out_shape=[
            jax.ShapeDtypeStruct((N, D), jnp.float32),
            jax.ShapeDtypeStruct((N, ROW), jnp.float32),
            jax.ShapeDtypeStruct((N, 1), jnp.float32),
        ],
    )(x, enc_W, enc_b, bn_g, bn_b, bn_m, bn_v, W, a_s, a_d)


def _fin_call(acc, bias, W, a_s, a_d):
    grid = (N // NBLK,)
    return pl.pallas_call(
        _fin_body,
        grid=grid,
        in_specs=[
            pl.BlockSpec((NBLK, ROW), lambda i: (i, 0)),
            _full2d((D,)),
            _full2d((D, D)), _full2d((D,)), _full2d((D,)),
        ],
        out_specs=[
            pl.BlockSpec((NBLK, ROW), lambda i: (i, 0)),
            pl.BlockSpec((NBLK, 1), lambda i: (i, 0)),
        ],
        out_shape=[
            jax.ShapeDtypeStruct((N, ROW), jnp.float32),
            jax.ShapeDtypeStruct((N, 1), jnp.float32),
        ],
    )(acc, bias, W, a_s, a_d)


def _last_call(acc, bias, x_init):
    grid = (N // NBLK,)
    return pl.pallas_call(
        _last_body,
        grid=grid,
        in_specs=[
            pl.BlockSpec((NBLK, ROW), lambda i: (i, 0)),
            _full2d((D,)),
            pl.BlockSpec((NBLK, D), lambda i: (i, 0)),
        ],
        out_specs=pl.BlockSpec((NBLK, D), lambda i: (i, 0)),
        out_shape=jax.ShapeDtypeStruct((N, D), jnp.float32),
    )(acc, bias, x_init)


def _el_call(ea, We, ae):
    grid = (E // EBLK,)
    return pl.pallas_call(
        _el_body,
        grid=grid,
        in_specs=[
            pl.BlockSpec((EBLK, 2), lambda i: (i, 0)),
            _full2d((2, D)), _full2d((D,)),
        ],
        out_specs=pl.BlockSpec((EBLK, 1), lambda i: (i, 0)),
        out_shape=jax.ShapeDtypeStruct((E, 1), jnp.float32),
    )(ea, We, ae)


def _sc_body(T_hbm, ad_hbm, ed_hbm, zeros_hbm, out_hbm,
             acc, ad_t, bufA, bufB, pend_src, pend_tgt, pend_p,
             tgt2, g_v, rows_v, semA, semB, sem, ssem):
    core = lax.axis_index("c")
    s = lax.axis_index("s")
    nslot = (NBUCKET + 1) // 2
    CW = 3 * CHUNK

    def ed_off(c):
        return (s * NCHUNK + c) * CW

    def gstart(f, p):
        pltpu.async_copy(
            T_hbm.at[pend_src.at[pl.ds(f * 128, 128)]], rows_v.at[p], sem)

    def gwait(f, p):
        pltpu.make_async_copy(
            T_hbm.at[pend_src.at[pl.ds(f * 128, 128)]],
            rows_v.at[p], sem).wait()

    def swait(p):
        pltpu.make_async_copy(rows_v.at[p], acc.at[tgt2.at[p]], ssem).wait()

    def burst(nb):
        @pl.when(nb > 0)
        def _prime():
            gstart(0, 0)

        def fire_body(f, carry):
            p = f & 1
            pn = (f + 1) & 1

            @pl.when(f >= 1)
            def _wsc():
                swait(pn)

            @pl.when(f + 1 < nb)
            def _gn():
                gstart(f + 1, pn)
            gwait(f, p)

            def gat_body(j, carry2):
                sl = pl.ds(j * 16, 16)
                pv = pend_p[pl.ds(f * 128 + j * 16, 16)]
                ridx = j * 16 + lax.iota(jnp.int32, 16)
                a_s = plsc.load_gather(
                    rows_v.at[p], [ridx, jnp.full((16,), D, jnp.int32)])
                logit = a_s + pv
                logit = jnp.where(logit >= 0, logit, 0.2 * logit)
                g_v[sl] = jnp.exp(logit)
                tgt2[p, sl] = pend_tgt[pl.ds(f * 128 + j * 16, 16)]
                return carry2
            lax.fori_loop(0, 8, gat_body, 0)

            def grp_body(j, carry2):
                gvec = g_v[pl.ds(j * 16, 16)]
                for k2 in range(16):
                    r = j * 16 + k2
                    gr = gvec[k2]
                    for cg in range(8):
                        csl = pl.ds(cg * 16, 16)
                        rows_v[p, r, csl] = rows_v[p, r, csl] * gr
                    one = lax.iota(jnp.int32, 16) == 0
                    rows_v[p, r, pl.ds(D, 16)] = jnp.where(one, gr, 0.0)
                return carry2
            lax.fori_loop(0, 8, grp_body, 0)

            pltpu.async_copy(rows_v.at[p], acc.at[tgt2.at[p]], ssem,
                             add=True)
            return carry
        lax.fori_loop(0, nb, fire_body, 0)

        @pl.when(nb > 0)
        def _drain():
            swait((nb - 1) & 1)

    def slot_body(slot, carry0):
        bucket = slot * 2 + core
        base = bucket * K

        @pl.when(bucket < NBUCKET)
        def _bucket_pass():
            @pl.when(s == 0)
            def _zero():
                pltpu.sync_copy(zeros_hbm, acc)
            pltpu.sync_copy(ad_hbm.at[pl.ds(base, K)], ad_t)
            plsc.subcore_barrier()

            def proc(buf, cur):
                def comp_body(j, cur2):
                    sv = buf[pl.ds(j * 16, 16)]
                    dv = buf[pl.ds(CHUNK + j * 16, 16)]
                    ev = plsc.bitcast(buf[pl.ds(2 * CHUNK + j * 16, 16)],
                                      jnp.float32)
                    inb = (dv >= base) & (dv < base + K)
                    loc = jnp.where(inb, dv - base, 0)
                    a_d = plsc.load_gather(ad_t, [loc])
                    part = a_d + ev
                    w = pl.ds(cur2, 16)
                    plsc.store_compressed(pend_src.at[w], sv, mask=inb)
                    plsc.store_compressed(pend_tgt.at[w], loc, mask=inb)
                    plsc.store_compressed(pend_p.at[w], part, mask=inb)
                    return cur2 + plsc.all_reduce_population_count(inb)[0]
                cur = lax.fori_loop(0, CHUNK // 16, comp_body, cur)

                nb = jnp.where(cur >= FIRE_T, cur // 128, 0)
                burst(nb)
                off = nb * 128
                for j2 in range(8):
                    sl_d = pl.ds(j2 * 16, 16)
                    sl_s = pl.ds(off + j2 * 16, 16)
                    pend_src[sl_d] = pend_src[sl_s]
                    pend_tgt[sl_d] = pend_tgt[sl_s]
                    pend_p[sl_d] = pend_p[sl_s]
                return cur - off

            pltpu.async_copy(ed_hbm.at[pl.ds(ed_off(0), CW)], bufA, semA)

            def pair_body(gp, cur):
                c1 = 2 * gp + 1

                @pl.when(c1 <= NCHUNK - 1)
                def _startB():
                    pltpu.async_copy(
                        ed_hbm.at[pl.ds(ed_off(c1), CW)], bufB, semB)
                pltpu.make_async_copy(
                    ed_hbm.at[pl.ds(ed_off(2 * gp), CW)], bufA, semA).wait()
                cur = proc(bufA, cur)

                @pl.when(2 * gp + 2 <= NCHUNK - 1)
                def _startA():
                    pltpu.async_copy(
                        ed_hbm.at[pl.ds(ed_off(2 * gp + 2), CW)], bufA, semA)

                def _doB():
                    pltpu.make_async_copy(
                        ed_hbm.at[pl.ds(ed_off(c1), CW)], bufB, semB).wait()
                    return proc(bufB, cur)
                return lax.cond(c1 <= NCHUNK - 1, _doB, lambda: cur)
            cur = lax.fori_loop(0, (NCHUNK + 1) // 2, pair_body, 0)

            def pad_body(j2, carry2):
                sl = pl.ds(j2 * 16, 16)
                lane = j2 * 16 + lax.iota(jnp.int32, 16)
                msk = lane < cur
                pend_src[sl] = jnp.where(msk, pend_src[sl], 0)
                pend_tgt[sl] = jnp.where(msk, pend_tgt[sl], K + s)
                pend_p[sl] = jnp.where(msk, pend_p[sl], NEG)
                return carry2
            lax.fori_loop(0, FIRE_T // 16, pad_body, 0)
            burst((cur + 127) // 128)
            plsc.subcore_barrier()

            per = K // NTILES
            r0 = s * per
            pltpu.sync_copy(acc.at[pl.ds(r0, per)],
                            out_hbm.at[pl.ds(base + r0, per)])
            plsc.subcore_barrier()
        return carry0
    lax.fori_loop(0, nslot, slot_body, 0)


def _sc_call(T, ad_full, edata, zeros):
    mesh = plsc.VectorSubcoreMesh(core_axis_name="c", subcore_axis_name="s",
                                  num_cores=2, num_subcores=NTILES)
    f = pl.kernel(
        _sc_body,
        out_type=jax.ShapeDtypeStruct((NPAD, ROW), jnp.float32),
        mesh=mesh,
        scratch_types=[
            pltpu.VMEM_SHARED((ACC_ROWS, ROW), jnp.float32),
            pltpu.VMEM((K,), jnp.float32),
            pltpu.VMEM((3 * CHUNK,), jnp.int32),
            pltpu.VMEM((3 * CHUNK,), jnp.int32),
            pltpu.VMEM((PEND,), jnp.int32),
            pltpu.VMEM((PEND,), jnp.int32),
            pltpu.VMEM((PEND,), jnp.float32),
            pltpu.VMEM((2, 128), jnp.int32),
            pltpu.VMEM((128,), jnp.float32),
            pltpu.VMEM((2, 128, ROW), jnp.float32),
            pltpu.SemaphoreType.DMA,
            pltpu.SemaphoreType.DMA,
            pltpu.SemaphoreType.DMA,
            pltpu.SemaphoreType.DMA,
        ],
        compiler_params=pltpu.CompilerParams(needs_layout_passes=False,
                                             use_tc_tiling_on_sc=False),
    )
    return f(T, ad_full, edata, zeros)


def kernel(x, edge_index, edge_attr, enc_W, enc_b, bn_g, bn_b, bn_m, bn_v,
           W1, as1, ad1, We1, ae1, b1,
           W2, as2, ad2, We2, ae2, b2,
           W3, as3, ad3, We3, ae3, b3):
    src = edge_index[0]
    dst = edge_index[1]
    pad = EPAD - E
    srcp = jnp.concatenate([src, jnp.zeros((pad,), jnp.int32)])
    dstp = jnp.concatenate([dst, jnp.full((pad,), N, jnp.int32)])
    src_r = srcp.reshape(NTILES, NCHUNK, CHUNK)
    dst_r = dstp.reshape(NTILES, NCHUNK, CHUNK)
    zeros = jnp.zeros((ACC_ROWS, ROW), jnp.float32)
    adz = jnp.zeros((NPAD - N,), jnp.float32)
    elz = jnp.zeros((pad,), jnp.float32)

    x_init, T, adcol = _enc_call(x, enc_W, enc_b, bn_g, bn_b, bn_m, bn_v,
                                 W1, as1, ad1)
    for (We, ae, bias, Wn, asn, adn) in (
            (We1, ae1, b1, W2, as2, ad2),
            (We2, ae2, b2, W3, as3, ad3),
            (We3, ae3, b3, None, None, None)):
        el = _el_call(edge_attr, We, ae)
        elp = jnp.concatenate([el[:, 0], elz])
        el_r = lax.bitcast_convert_type(elp, jnp.int32).reshape(
            NTILES, NCHUNK, CHUNK)
        edata = jnp.stack([src_r, dst_r, el_r], axis=2).reshape(-1)
        ad_full = jnp.concatenate([adcol[:, 0], adz])
        acc = _sc_call(T, ad_full, edata, zeros)
        if Wn is None:
            return _last_call(acc, bias, x_init)
        T, adcol = _fin_call(acc, bias, Wn, asn, adn)


# speedup vs baseline: 1.4086x; 1.4086x over previous
import jax
import jax.numpy as jnp
from jax import lax
from jax.experimental import pallas as pl
from jax.experimental.pallas import tpu as pltpu
from jax.experimental.pallas import tpu_sc as plsc

N = 100000
E = 1600000
D = 128
ROW = 144
K = 7168
NBUCKET = 14
NPAD = NBUCKET * K
NTILES = 16
CHUNK = 1024
NCHUNK = 98
EPT = NCHUNK * CHUNK
EPAD = NTILES * EPT
ACC_ROWS = K + 16
PEND = 2176
FIRE_T = 1024
NEG = -1e30
NBLK = 2000
EBLK = 8000


def _enc_body(x_ref, encW_ref, encb_ref, bg_ref, bb_ref, bm_ref, bv_ref,
              W_ref, as_ref, ad_ref, xinit_ref, T_ref, adcol_ref):
    h = jnp.dot(x_ref[...], encW_ref[...], preferred_element_type=jnp.float32)
    h = h + encb_ref[...]
    h = (h - bm_ref[...]) / jnp.sqrt(bv_ref[...] + 1e-5) * bg_ref[...] + bb_ref[...]
    h = jnp.maximum(h, 0.0)
    xinit_ref[...] = h
    xt = jnp.dot(h, W_ref[...], preferred_element_type=jnp.float32)
    T_ref[:, :D] = xt
    a_s = jnp.dot(xt, as_ref[...], preferred_element_type=jnp.float32)
    T_ref[:, D:ROW] = jnp.broadcast_to(a_s[:, None], (NBLK, ROW - D))
    adcol_ref[...] = jnp.dot(xt, ad_ref[...], preferred_element_type=jnp.float32)[:, None]


def _fin_body(acc_ref, bias_ref, W_ref, as_ref, ad_ref, T_ref, adcol_ref):
    den = acc_ref[:, D:D + 1] + 1e-16
    h = acc_ref[:, :D] / den + bias_ref[...]
    h = jnp.maximum(h, 0.0)
    xt = jnp.dot(h, W_ref[...], preferred_element_type=jnp.float32)
    T_ref[:, :D] = xt
    a_s = jnp.dot(xt, as_ref[...], preferred_element_type=jnp.float32)
    T_ref[:, D:ROW] = jnp.broadcast_to(a_s[:, None], (NBLK, ROW - D))
    adcol_ref[...] = jnp.dot(xt, ad_ref[...], preferred_element_type=jnp.float32)[:, None]


def _last_body(acc_ref, bias_ref, xinit_ref, out_ref):
    den = acc_ref[:, D:D + 1] + 1e-16
    out_ref[...] = acc_ref[:, :D] / den + bias_ref[...] + xinit_ref[...]


def _el_body(ea_ref, We_ref, ae_ref, el_ref):
    m = jnp.dot(We_ref[...], ae_ref[...], preferred_element_type=jnp.float32)
    el_ref[...] = jnp.dot(ea_ref[...], m, preferred_element_type=jnp.float32)[:, None]


def _full2d(shape):
    return pl.BlockSpec(shape, lambda i: (0,) * len(shape))


def _enc_call(x, enc_W, enc_b, bn_g, bn_b, bn_m, bn_v, W, a_s, a_d):
    grid = (N // NBLK,)
    return pl.pallas_call(
        _enc_body,
        grid=grid,
        in_specs=[
            pl.BlockSpec((NBLK, 8), lambda i: (i, 0)),
            _full2d((8, D)), _full2d((D,)), _full2d((D,)), _full2d((D,)),
            _full2d((D,)), _full2d((D,)),
            _full2d((D, D)), _full2d((D,)), _full2d((D,)),
        ],
        out_specs=[
            pl.BlockSpec((NBLK, D), lambda i: (i, 0)),
            pl.BlockSpec((NBLK, ROW), lambda i: (i, 0)),
            pl.BlockSpec((NBLK, 1), lambda i: (i, 0)),
        ],
        out_shape=[
            jax.ShapeDtypeStruct((N, D), jnp.float32),
            jax.ShapeDtypeStruct((N, ROW), jnp.float32),
            jax.ShapeDtypeStruct((N, 1), jnp.float32),
        ],
    )(x, enc_W, enc_b, bn_g, bn_b, bn_m, bn_v, W, a_s, a_d)


def _fin_call(acc, bias, W, a_s, a_d):
    grid = (N // NBLK,)
    return pl.pallas_call(
        _fin_body,
        grid=grid,
        in_specs=[
            pl.BlockSpec((NBLK, ROW), lambda i: (i, 0)),
            _full2d((D,)),
            _full2d((D, D)), _full2d((D,)), _full2d((D,)),
        ],
        out_specs=[
            pl.BlockSpec((NBLK, ROW), lambda i: (i, 0)),
            pl.BlockSpec((NBLK, 1), lambda i: (i, 0)),
        ],
        out_shape=[
            jax.ShapeDtypeStruct((N, ROW), jnp.float32),
            jax.ShapeDtypeStruct((N, 1), jnp.float32),
        ],
    )(acc, bias, W, a_s, a_d)


def _last_call(acc, bias, x_init):
    grid = (N // NBLK,)
    return pl.pallas_call(
        _last_body,
        grid=grid,
        in_specs=[
            pl.BlockSpec((NBLK, ROW), lambda i: (i, 0)),
            _full2d((D,)),
            pl.BlockSpec((NBLK, D), lambda i: (i, 0)),
        ],
        out_specs=pl.BlockSpec((NBLK, D), lambda i: (i, 0)),
        out_shape=jax.ShapeDtypeStruct((N, D), jnp.float32),
    )(acc, bias, x_init)


def _el_call(ea, We, ae):
    grid = (E // EBLK,)
    return pl.pallas_call(
        _el_body,
        grid=grid,
        in_specs=[
            pl.BlockSpec((EBLK, 2), lambda i: (i, 0)),
            _full2d((2, D)), _full2d((D,)),
        ],
        out_specs=pl.BlockSpec((EBLK, 1), lambda i: (i, 0)),
        out_shape=jax.ShapeDtypeStruct((E, 1), jnp.float32),
    )(ea, We, ae)


def _sc_body(T_hbm, ad_hbm, ed_hbm, zeros_hbm, out_hbm,
             acc, ad_t, bufA, bufB, pend_src, pend_tgt, pend_p,
             tgt2, g_v, rows_v, semA, semB, sem, ssem):
    core = lax.axis_index("c")
    s = lax.axis_index("s")
    nslot = (NBUCKET + 1) // 2
    CW = 3 * CHUNK

    def ed_off(c):
        return (s * NCHUNK + c) * CW

    def gstart(f, p):
        pltpu.async_copy(
            T_hbm.at[pend_src.at[pl.ds(f * 128, 128)]], rows_v.at[p], sem)

    def gwait(f, p):
        pltpu.make_async_copy(
            T_hbm.at[pend_src.at[pl.ds(f * 128, 128)]],
            rows_v.at[p], sem).wait()

    def swait(p):
        pltpu.make_async_copy(rows_v.at[p], acc.at[tgt2.at[p]], ssem).wait()

    def burst(nb):
        @pl.when(nb > 0)
        def _prime():
            gstart(0, 0)

        def fire_body(f, carry):
            p = f & 1
            pn = (f + 1) & 1

            @pl.when(f >= 1)
            def _wsc():
                swait(pn)

            @pl.when(f + 1 < nb)
            def _gn():
                gstart(f + 1, pn)
            gwait(f, p)

            def fire_compute(pi):
                def gat_body(j, carry2):
                    sl = pl.ds(j * 16, 16)
                    pv = pend_p[pl.ds(f * 128 + j * 16, 16)]
                    ridx = j * 16 + lax.iota(jnp.int32, 16)
                    a_s = plsc.load_gather(
                        rows_v.at[pi], [ridx, jnp.full((16,), D, jnp.int32)])
                    logit = a_s + pv
                    logit = jnp.where(logit >= 0, logit, 0.2 * logit)
                    g_v[sl] = jnp.exp(logit)
                    tgt2[pi, sl] = pend_tgt[pl.ds(f * 128 + j * 16, 16)]
                    return carry2
                lax.fori_loop(0, 8, gat_body, 0)

                def grp_body(j, carry2):
                    gvec = g_v[pl.ds(j * 16, 16)]
                    for k2 in range(16):
                        r = j * 16 + k2
                        gr = gvec[k2]
                        for cg in range(8):
                            csl = pl.ds(cg * 16, 16)
                            rows_v[pi, r, csl] = rows_v[pi, r, csl] * gr
                        one = lax.iota(jnp.int32, 16) == 0
                        rows_v[pi, r, pl.ds(D, 16)] = jnp.where(one, gr, 0.0)
                    return carry2
                lax.fori_loop(0, 8, grp_body, 0)

                pltpu.async_copy(rows_v.at[pi], acc.at[tgt2.at[pi]], ssem,
                                 add=True)

            @pl.when(p == 0)
            def _c0():
                fire_compute(0)

            @pl.when(p == 1)
            def _c1():
                fire_compute(1)
            return carry
        lax.fori_loop(0, nb, fire_body, 0)

        @pl.when(nb > 0)
        def _drain():
            swait((nb - 1) & 1)

    def slot_body(slot, carry0):
        bucket = slot * 2 + core
        base = bucket * K

        @pl.when(bucket < NBUCKET)
        def _bucket_pass():
            @pl.when(s == 0)
            def _zero():
                pltpu.sync_copy(zeros_hbm, acc)
            pltpu.sync_copy(ad_hbm.at[pl.ds(base, K)], ad_t)
            plsc.subcore_barrier()

            def proc(buf, cur):
                def comp_body(j, cur2):
                    sv = buf[pl.ds(j * 16, 16)]
                    dv = buf[pl.ds(CHUNK + j * 16, 16)]
                    ev = plsc.bitcast(buf[pl.ds(2 * CHUNK + j * 16, 16)],
                                      jnp.float32)
                    inb = (dv >= base) & (dv < base + K)
                    loc = jnp.where(inb, dv - base, 0)
                    a_d = plsc.load_gather(ad_t, [loc])
                    part = a_d + ev
                    w = pl.ds(cur2, 16)
                    plsc.store_compressed(pend_src.at[w], sv, mask=inb)
                    plsc.store_compressed(pend_tgt.at[w], loc, mask=inb)
                    plsc.store_compressed(pend_p.at[w], part, mask=inb)
                    return cur2 + plsc.all_reduce_population_count(inb)[0]
                cur = lax.fori_loop(0, CHUNK // 16, comp_body, cur)

                nb = jnp.where(cur >= FIRE_T, cur // 128, 0)
                burst(nb)
                off = nb * 128
                for j2 in range(8):
                    sl_d = pl.ds(j2 * 16, 16)
                    sl_s = pl.ds(off + j2 * 16, 16)
                    pend_src[sl_d] = pend_src[sl_s]
                    pend_tgt[sl_d] = pend_tgt[sl_s]
                    pend_p[sl_d] = pend_p[sl_s]
                return cur - off

            pltpu.async_copy(ed_hbm.at[pl.ds(ed_off(0), CW)], bufA, semA)

            def pair_body(gp, cur):
                c1 = 2 * gp + 1

                @pl.when(c1 <= NCHUNK - 1)
                def _startB():
                    pltpu.async_copy(
                        ed_hbm.at[pl.ds(ed_off(c1), CW)], bufB, semB)
                pltpu.make_async_copy(
                    ed_hbm.at[pl.ds(ed_off(2 * gp), CW)], bufA, semA).wait()
                cur = proc(bufA, cur)

                @pl.when(2 * gp + 2 <= NCHUNK - 1)
                def _startA():
                    pltpu.async_copy(
                        ed_hbm.at[pl.ds(ed_off(2 * gp + 2), CW)], bufA, semA)

                def _doB():
                    pltpu.make_async_copy(
                        ed_hbm.at[pl.ds(ed_off(c1), CW)], bufB, semB).wait()
                    return proc(bufB, cur)
                return lax.cond(c1 <= NCHUNK - 1, _doB, lambda: cur)
            cur = lax.fori_loop(0, (NCHUNK + 1) // 2, pair_body, 0)

            def pad_body(j2, carry2):
                sl = pl.ds(j2 * 16, 16)
                lane = j2 * 16 + lax.iota(jnp.int32, 16)
                msk = lane < cur
                pend_src[sl] = jnp.where(msk, pend_src[sl], 0)
                pend_tgt[sl] = jnp.where(msk, pend_tgt[sl], K + s)
                pend_p[sl] = jnp.where(msk, pend_p[sl], NEG)
                return carry2
            lax.fori_loop(0, FIRE_T // 16, pad_body, 0)
            burst((cur + 127) // 128)
            plsc.subcore_barrier()

            per = K // NTILES
            r0 = s * per
            pltpu.sync_copy(acc.at[pl.ds(r0, per)],
                            out_hbm.at[pl.ds(base + r0, per)])
            plsc.subcore_barrier()
        return carry0
    lax.fori_loop(0, nslot, slot_body, 0)


def _sc_call(T, ad_full, edata, zeros):
    mesh = plsc.VectorSubcoreMesh(core_axis_name="c", subcore_axis_name="s",
                                  num_cores=2, num_subcores=NTILES)
    f = pl.kernel(
        _sc_body,
        out_type=jax.ShapeDtypeStruct((NPAD, ROW), jnp.float32),
        mesh=mesh,
        scratch_types=[
            pltpu.VMEM_SHARED((ACC_ROWS, ROW), jnp.float32),
            pltpu.VMEM((K,), jnp.float32),
            pltpu.VMEM((3 * CHUNK,), jnp.int32),
            pltpu.VMEM((3 * CHUNK,), jnp.int32),
            pltpu.VMEM((PEND,), jnp.int32),
            pltpu.VMEM((PEND,), jnp.int32),
            pltpu.VMEM((PEND,), jnp.float32),
            pltpu.VMEM((2, 128), jnp.int32),
            pltpu.VMEM((128,), jnp.float32),
            pltpu.VMEM((2, 128, ROW), jnp.float32),
            pltpu.SemaphoreType.DMA,
            pltpu.SemaphoreType.DMA,
            pltpu.SemaphoreType.DMA,
            pltpu.SemaphoreType.DMA,
        ],
        compiler_params=pltpu.CompilerParams(needs_layout_passes=False,
                                             use_tc_tiling_on_sc=False),
    )
    return f(T, ad_full, edata, zeros)


def kernel(x, edge_index, edge_attr, enc_W, enc_b, bn_g, bn_b, bn_m, bn_v,
           W1, as1, ad1, We1, ae1, b1,
           W2, as2, ad2, We2, ae2, b2,
           W3, as3, ad3, We3, ae3, b3):
    src = edge_index[0]
    dst = edge_index[1]
    pad = EPAD - E
    srcp = jnp.concatenate([src, jnp.zeros((pad,), jnp.int32)])
    dstp = jnp.concatenate([dst, jnp.full((pad,), N, jnp.int32)])
    src_r = srcp.reshape(NTILES, NCHUNK, CHUNK)
    dst_r = dstp.reshape(NTILES, NCHUNK, CHUNK)
    zeros = jnp.zeros((ACC_ROWS, ROW), jnp.float32)
    adz = jnp.zeros((NPAD - N,), jnp.float32)
    elz = jnp.zeros((pad,), jnp.float32)

    x_init, T, adcol = _enc_call(x, enc_W, enc_b, bn_g, bn_b, bn_m, bn_v,
                                 W1, as1, ad1)
    for (We, ae, bias, Wn, asn, adn) in (
            (We1, ae1, b1, W2, as2, ad2),
            (We2, ae2, b2, W3, as3, ad3),
            (We3, ae3, b3, None, None, None)):
        el = _el_call(edge_attr, We, ae)
        elp = jnp.concatenate([el[:, 0], elz])
        el_r = lax.bitcast_convert_type(elp, jnp.int32).reshape(
            NTILES, NCHUNK, CHUNK)
        edata = jnp.stack([src_r, dst_r, el_r], axis=2).reshape(-1)
        ad_full = jnp.concatenate([adcol[:, 0], adz])
        acc = _sc_call(T, ad_full, edata, zeros)
        if Wn is None:
            return _last_call(acc, bias, x_init)
        T, adcol = _fin_call(acc, bias, Wn, asn, adn)
